# Initial kernel scaffold; baseline (speedup 1.0000x reference)
import jax
import jax.numpy as jnp
from jax.experimental import pallas as pl


def kernel(x, bases0, wcomp0, sl0, bases1, wcomp1, sl1, edge_index, edge_type):
    raise NotImplementedError("write your pallas kernel here")


# speedup vs baseline: 2.2952x; 2.2952x over previous
import functools

import jax
import jax.numpy as jnp
from jax import lax
from jax.experimental import pallas as pl
from jax.experimental.pallas import tpu as pltpu
from jax.experimental.pallas import tpu_sc as plsc

N = 10000
E = 320000
IN = 128
H = 32
R = 51
NB = 4
RP = R + 1

NTILES = 32
CHUNK = 128
NSTEP = 82
EP = NTILES * NSTEP * CHUNK
NP = 10112
RPT = NP // 16
BN = 1024
NBLK = 10


def _table0_body(wc_ref, x_ref, b_ref, o_ref):
    r = pl.program_id(1)
    relw = (wc_ref[r, 0] * b_ref[0 * IN:1 * IN]
            + wc_ref[r, 1] * b_ref[1 * IN:2 * IN]
            + wc_ref[r, 2] * b_ref[2 * IN:3 * IN]
            + wc_ref[r, 3] * b_ref[3 * IN:4 * IN]
            + wc_ref[r, 4] * b_ref[4 * IN:5 * IN])
    o_ref[0] = jnp.dot(x_ref[...], relw, preferred_element_type=jnp.float32)


def _table1_body(wc_ref, m0_ref, m1_ref, b_ref, o_ref):
    r = pl.program_id(1)
    relw = (wc_ref[r, 0] * b_ref[0 * H:1 * H]
            + wc_ref[r, 1] * b_ref[1 * H:2 * H]
            + wc_ref[r, 2] * b_ref[2 * H:3 * H]
            + wc_ref[r, 3] * b_ref[3 * H:4 * H]
            + wc_ref[r, 4] * b_ref[4 * H:5 * H])
    h = jnp.maximum(m0_ref[...] + m1_ref[...], 0.0)
    o_ref[0] = jnp.dot(h, relw, preferred_element_type=jnp.float32)


def _relu_body(m0_ref, m1_ref, o_ref):
    o_ref[...] = jnp.maximum(m0_ref[...] + m1_ref[...], 0.0)


_table0 = pl.pallas_call(
    _table0_body,
    grid=(NBLK, RP),
    in_specs=[
        pl.BlockSpec(memory_space=pltpu.SMEM),
        pl.BlockSpec((BN, IN), lambda n, r: (n, 0)),
        pl.BlockSpec((5 * IN, H), lambda n, r: (0, 0)),
    ],
    out_specs=pl.BlockSpec((1, BN, H), lambda n, r: (r, n, 0)),
    out_shape=jax.ShapeDtypeStruct((RP, N, H), jnp.float32),
)

_table1 = pl.pallas_call(
    _table1_body,
    grid=(NBLK, RP),
    in_specs=[
        pl.BlockSpec(memory_space=pltpu.SMEM),
        pl.BlockSpec((BN, H), lambda n, r: (n, 0)),
        pl.BlockSpec((BN, H), lambda n, r: (n, 0)),
        pl.BlockSpec((5 * H, H), lambda n, r: (0, 0)),
    ],
    out_specs=pl.BlockSpec((1, BN, H), lambda n, r: (r, n, 0)),
    out_shape=jax.ShapeDtypeStruct((RP, N, H), jnp.float32),
)

_final_relu = pl.pallas_call(
    _relu_body,
    grid=(NBLK,),
    in_specs=[
        pl.BlockSpec((BN, H), lambda n: (n, 0)),
        pl.BlockSpec((BN, H), lambda n: (n, 0)),
    ],
    out_specs=pl.BlockSpec((BN, H), lambda n: (n, 0)),
    out_shape=jax.ShapeDtypeStruct((N, H), jnp.float32),
)


def _edge_body(t_ref, g_ref, d_ref, z_ref, out_ref,
               gv, dv, buf0, buf1, obuf, mail, sem0, sem1):
    c = lax.axis_index("c")
    s = lax.axis_index("s")
    wid = c * 16 + s
    pltpu.sync_copy(g_ref.at[wid], gv)
    pltpu.sync_copy(d_ref.at[wid], dv)
    pltpu.sync_copy(z_ref.at[pl.ds(s * RPT, RPT)], mail.at[pl.ds(s * RPT, RPT)])
    plsc.subcore_barrier()

    pltpu.async_copy(t_ref.at[gv.at[0]], buf0, sem0)

    def step(j, carry):
        i0 = 2 * j
        pltpu.async_copy(t_ref.at[gv.at[i0 + 1]], buf1, sem1)
        pltpu.make_async_copy(t_ref.at[pl.ds(0, CHUNK)], buf0, sem0).wait()
        pltpu.sync_copy(buf0, mail.at[dv.at[i0]], add=True)

        @pl.when(i0 + 2 < NSTEP)
        def _():
            pltpu.async_copy(t_ref.at[gv.at[i0 + 2]], buf0, sem0)

        pltpu.make_async_copy(t_ref.at[pl.ds(0, CHUNK)], buf1, sem1).wait()
        pltpu.sync_copy(buf1, mail.at[dv.at[i0 + 1]], add=True)
        return carry

    lax.fori_loop(0, NSTEP // 2, step, 0)
    plsc.subcore_barrier()
    pltpu.sync_copy(mail.at[pl.ds(s * RPT, RPT)], obuf)
    pltpu.sync_copy(obuf, out_ref.at[c, pl.ds(s * RPT, RPT)])


@functools.cache
def _edge_pass():
    return pl.kernel(
        _edge_body,
        out_type=jax.ShapeDtypeStruct((2, NP, H), jnp.float32),
        mesh=plsc.VectorSubcoreMesh(core_axis_name="c", subcore_axis_name="s"),
        scratch_types=[
            pltpu.VMEM((NSTEP, CHUNK), jnp.int32),
            pltpu.VMEM((NSTEP, CHUNK), jnp.int32),
            pltpu.VMEM((CHUNK, H), jnp.float32),
            pltpu.VMEM((CHUNK, H), jnp.float32),
            pltpu.VMEM((RPT, H), jnp.float32),
            pltpu.VMEM_SHARED((NP, H), jnp.float32),
            pltpu.SemaphoreType.DMA,
            pltpu.SemaphoreType.DMA,
        ],
        compiler_params=pltpu.CompilerParams(use_tc_tiling_on_sc=False),
    )


def kernel(x, bases0, wcomp0, sl0, bases1, wcomp1, sl1, edge_index, edge_type):
    src = edge_index[0]
    dst = edge_index[1]
    wc0 = jnp.zeros((RP, 8), jnp.float32).at[:R, :NB].set(wcomp0).at[R, NB].set(1.0)
    wc1 = jnp.zeros((RP, 8), jnp.float32).at[:R, :NB].set(wcomp1).at[R, NB].set(1.0)
    b0 = jnp.concatenate([bases0, sl0[None]], axis=0).reshape(5 * IN, H)
    b1 = jnp.concatenate([bases1, sl1[None]], axis=0).reshape(5 * H, H)

    ar_n = jnp.arange(N, dtype=jnp.int32)
    g = jnp.concatenate([edge_type * N + src, R * N + ar_n])
    d = jnp.concatenate([dst, ar_n])
    npad = EP - (E + N)
    ppad = jnp.arange(npad, dtype=jnp.int32)
    gpad = (ppad * 1009) % (RP * N)
    dpad = N + (ppad % (NP - N))
    g = jnp.concatenate([g, gpad]).reshape(NTILES, NSTEP, CHUNK)
    d = jnp.concatenate([d, dpad]).reshape(NTILES, NSTEP, CHUNK)
    zeros = jnp.zeros((NP, H), jnp.float32)

    edge_pass = _edge_pass()
    t0 = _table0(wc0, x, b0)
    m0 = edge_pass(t0.reshape(RP * N, H), g, d, zeros)
    t1 = _table1(wc1, m0[0, :N], m0[1, :N], b1)
    m1 = edge_pass(t1.reshape(RP * N, H), g, d, zeros)
    return _final_relu(m1[0, :N], m1[1, :N])
